# baseline (speedup 1.0000x reference)
import functools

import jax
import jax.numpy as jnp
from jax import lax
from jax.experimental import pallas as pl
from jax.experimental.pallas import tpu as pltpu
from jax.experimental.pallas import tpu_sc as plsc

_LANES = 16
_CHUNK = 128


def _make_sc_kernel(B, V, D, n_workers, num_cores):
    b_per_w = B // n_workers
    n_chunks = b_per_w // _CHUNK
    w = 2 * D
    mesh = plsc.VectorSubcoreMesh(core_axis_name="c", subcore_axis_name="s")

    @functools.partial(
        pl.kernel,
        out_type=jax.ShapeDtypeStruct((B,), jnp.float32),
        mesh=mesh,
        scratch_types=[
            pltpu.VMEM((b_per_w,), jnp.int32),
            pltpu.VMEM((b_per_w,), jnp.int32),
            pltpu.VMEM((b_per_w,), jnp.int32),
            pltpu.VMEM((b_per_w,), jnp.int32),
            pltpu.VMEM((2, _CHUNK, w), jnp.float32),
            pltpu.VMEM((2, _CHUNK, w), jnp.float32),
            pltpu.VMEM((b_per_w,), jnp.float32),
            pltpu.SemaphoreType.DMA,
            pltpu.SemaphoreType.DMA,
        ],
        compiler_params=pltpu.CompilerParams(needs_layout_passes=False),
    )
    def sc_kernel(center_hbm, context_hbm, in2_hbm, out2_hbm, score_hbm,
                  cidx, xidx, cq, xq, crows, xrows, scores, sem0, sem1):
        wid = lax.axis_index("s") * num_cores + lax.axis_index("c")
        base = wid * b_per_w
        pltpu.sync_copy(center_hbm.at[pl.ds(base, b_per_w)], cidx)
        pltpu.sync_copy(context_hbm.at[pl.ds(base, b_per_w)], xidx)

        def halve(i, carry):
            sl = pl.ds(i * _LANES, _LANES)
            cq[sl] = lax.shift_right_logical(cidx[sl], 1)
            xq[sl] = lax.shift_right_logical(xidx[sl], 1)
            return carry

        lax.fori_loop(0, b_per_w // _LANES, halve, 0)

        sems = (sem0, sem1)

        def fire(k):
            sl = pl.ds(k * _CHUNK, _CHUNK)
            sem = sems[k % 2]
            pltpu.async_copy(in2_hbm.at[cq.at[sl]], crows.at[k % 2], sem)
            pltpu.async_copy(out2_hbm.at[xq.at[sl]], xrows.at[k % 2], sem)

        def drain(k):
            sem = sems[k % 2]
            pltpu.make_async_copy(in2_hbm.at[pl.ds(0, _CHUNK)],
                                  crows.at[k % 2], sem).wait()
            pltpu.make_async_copy(out2_hbm.at[pl.ds(0, _CHUNK)],
                                  xrows.at[k % 2], sem).wait()

        def make_compute(k):
            def compute(g, carry):
                rows = g * _LANES + lax.iota(jnp.int32, _LANES)
                tok = pl.ds(k * _CHUNK + g * _LANES, _LANES)
                cbase = (cidx[tok] & 1) * D
                xbase = (xidx[tok] & 1) * D
                acc = jnp.zeros((_LANES,), jnp.float32)
                for d in range(D):
                    gc = plsc.load_gather(crows.at[k % 2], [rows, cbase + d])
                    gx = plsc.load_gather(xrows.at[k % 2], [rows, xbase + d])
                    acc = acc + gc * gx
                scores[tok] = acc
                return carry
            return compute

        fire(0)
        for k in range(n_chunks):
            if k + 1 < n_chunks:
                fire(k + 1)
            drain(k)
            lax.fori_loop(0, _CHUNK // _LANES, make_compute(k), 0)

        pltpu.sync_copy(scores, score_hbm.at[pl.ds(base, b_per_w)])

    return sc_kernel


def kernel(center, context, in_table, out_table):
    B, = center.shape
    V, D = in_table.shape
    info = plsc.get_sparse_core_info()
    n_workers = info.num_cores * info.num_subcores
    sc_kernel = _make_sc_kernel(B, V, D, n_workers, info.num_cores)
    in2 = in_table.reshape(V // 2, 2 * D)
    out2 = out_table.reshape(V // 2, 2 * D)
    return sc_kernel(center, context, in2, out2)
